# baseline (speedup 1.0000x reference)
import functools

import jax
import jax.numpy as jnp
from jax import lax
from jax.experimental import pallas as pl
from jax.experimental.pallas import tpu as pltpu
from jax.experimental.pallas import tpu_sc as plsc

E_DIM = 64
CHUNK = 128


@functools.cache
def _build(B: int):
    info = plsc.get_sparse_core_info()
    nw = info.num_cores * info.num_subcores
    b_w = B // nw
    n_chunks = b_w // CHUNK
    mesh = plsc.VectorSubcoreMesh(core_axis_name="c", subcore_axis_name="s")

    @functools.partial(
        pl.kernel,
        out_type=jax.ShapeDtypeStruct((B, E_DIM), jnp.float32),
        mesh=mesh,
        scratch_types=[
            pltpu.VMEM((CHUNK,), jnp.int32),
            pltpu.VMEM((CHUNK, E_DIM), jnp.float32),
            pltpu.SemaphoreType.DMA,
        ],
        compiler_params=pltpu.CompilerParams(use_tc_tiling_on_sc=False),
    )
    def emb(x_hbm, w_hbm, out_hbm, idx_v, rows_v, sem):
        wid = lax.axis_index("s") * info.num_cores + lax.axis_index("c")
        base = wid * b_w

        @pl.loop(0, n_chunks)
        def _chunk(g):
            off = base + g * CHUNK
            pltpu.sync_copy(x_hbm.at[pl.ds(off, CHUNK)], idx_v)
            pltpu.async_copy(w_hbm.at[idx_v], rows_v, sem).wait()
            pltpu.sync_copy(rows_v, out_hbm.at[pl.ds(off, CHUNK)])

    return emb


@jax.jit
def kernel(X, W):
    batch, seq = X.shape
    flat_idx = X.reshape(batch * seq).astype(jnp.int32)
    out = _build(batch * seq)(flat_idx, W)
    return out.reshape(batch, seq, E_DIM)


# speedup vs baseline: 1.1866x; 1.1866x over previous
import functools

import jax
import jax.numpy as jnp
from jax import lax
from jax.experimental import pallas as pl
from jax.experimental.pallas import tpu as pltpu
from jax.experimental.pallas import tpu_sc as plsc

E_DIM = 64
STREAM = 128
SUP = 512
K = SUP // STREAM
NBUF = 2


@functools.cache
def _build(B: int):
    info = plsc.get_sparse_core_info()
    nw = info.num_cores * info.num_subcores
    b_w = B // nw
    n_sup = b_w // SUP
    mesh = plsc.VectorSubcoreMesh(core_axis_name="c", subcore_axis_name="s")

    @functools.partial(
        pl.kernel,
        out_type=jax.ShapeDtypeStruct((B, E_DIM), jnp.float32),
        mesh=mesh,
        scratch_types=[
            pltpu.VMEM((NBUF, SUP), jnp.int32),
            pltpu.VMEM((NBUF, SUP, E_DIM), jnp.float32),
            pltpu.SemaphoreType.DMA((NBUF,)),
        ],
        compiler_params=pltpu.CompilerParams(use_tc_tiling_on_sc=False),
    )
    def emb(x_hbm, w_hbm, out_hbm, idx_v, rows_v, gsem):
        wid = lax.axis_index("s") * info.num_cores + lax.axis_index("c")
        base = wid * b_w

        def issue(b, g):
            off = base + g * SUP
            pltpu.sync_copy(x_hbm.at[pl.ds(off, SUP)], idx_v.at[b])
            for k in range(K):
                pltpu.async_copy(
                    w_hbm.at[idx_v.at[b, pl.ds(k * STREAM, STREAM)]],
                    rows_v.at[b, pl.ds(k * STREAM, STREAM)],
                    gsem.at[b],
                )

        def drain(b):
            pltpu.make_async_copy(
                w_hbm.at[pl.ds(0, SUP)], rows_v.at[b], gsem.at[b]
            ).wait()

        issue(0, 0)

        @pl.loop(0, n_sup, step=NBUF)
        def _outer(g0):
            for b in range(NBUF):
                g = g0 + b
                nb = (b + 1) % NBUF

                @pl.when(g + 1 < n_sup)
                def _prefetch():
                    issue(nb, g + 1)

                drain(b)
                pltpu.sync_copy(rows_v.at[b], out_hbm.at[pl.ds(base + g * SUP, SUP)])

    return emb


@jax.jit
def kernel(X, W):
    batch, seq = X.shape
    flat_idx = X.reshape(batch * seq).astype(jnp.int32)
    out = _build(batch * seq)(flat_idx, W)
    return out.reshape(batch, seq, E_DIM)
